# baseline (speedup 1.0000x reference)
import functools

import jax
import jax.numpy as jnp
from jax import lax
from jax.experimental import pallas as pl
from jax.experimental.pallas import tpu as pltpu
from jax.experimental.pallas import tpu_sc as plsc

_K = 20
_EPS = 1e-5
_SLOPE = 0.2
_NEG = -3.0e38


def _topk_kernel(xt_rows_ref, x_full_ref, w1t_ref, wd_ref,
                 idx_ref, gt_ref, ht_ref):
    b = pl.program_id(0)

    x_b = x_full_ref[0]
    rows = xt_rows_ref[0]
    R = rows.shape[0]
    N = x_b.shape[1]

    d2 = lax.dot_general(rows, x_b, (((1,), (0,)), ((), ())),
                         preferred_element_type=jnp.float32)
    xxr = jnp.sum(rows * rows, axis=1, keepdims=True)
    xxc = jnp.sum(x_b * x_b, axis=0, keepdims=True)
    negd = 2.0 * d2 - xxr - xxc

    lane = lax.broadcasted_iota(jnp.int32, (R, N), 1)

    ams = []
    for _ in range(_K):
        m = jnp.max(negd, axis=1, keepdims=True)
        cand = jnp.where(negd == m, lane, N)
        am = jnp.min(cand, axis=1, keepdims=True)
        ams.append(am)
        negd = jnp.where(lane == am, _NEG, negd)

    idx_tile = jnp.concatenate(ams, axis=1) + b * N
    idx_ref[...] = idx_tile[None]

    gt_ref[...] = lax.dot_general(rows, w1t_ref[...], (((1,), (0,)), ((), ())),
                                  preferred_element_type=jnp.float32)[None]
    ht_ref[...] = lax.dot_general(rows, wd_ref[...], (((1,), (0,)), ((), ())),
                                  preferred_element_type=jnp.float32)[None]


def _bn_kernel(pmax_ref, pmin_ref, scale_ref, shift_ref, out_ref):
    scale = scale_ref[...]
    shift = shift_ref[...]
    pick = jnp.where(scale >= 0.0, pmax_ref[0], pmin_ref[0])
    z = pick * scale + shift
    out_ref[...] = jnp.where(z >= 0.0, z, _SLOPE * z)[None]


def _make_gather_reduce(P, Co, NC, NS):
    NW = NC * NS
    PTS_W = P // NW
    CH = 64
    NCH = PTS_W // CH
    ROWS = CH * _K
    IDXR = ROWS // 128
    WIDXR = PTS_W * _K // 128
    CSL = Co // 16

    mesh = plsc.VectorSubcoreMesh(core_axis_name="c", subcore_axis_name="s")

    @functools.partial(
        pl.kernel, mesh=mesh,
        compiler_params=pltpu.CompilerParams(use_tc_tiling_on_sc=False),
        out_type=[
            jax.ShapeDtypeStruct((P, Co), jnp.float32),
            jax.ShapeDtypeStruct((P, Co), jnp.float32),
            jax.ShapeDtypeStruct((NW, CSL, 16), jnp.float32),
            jax.ShapeDtypeStruct((NW, CSL, 16), jnp.float32),
        ],
        scratch_types=[
            pltpu.VMEM((WIDXR, 128), jnp.int32),
            pltpu.VMEM((ROWS, Co), jnp.float32),
            pltpu.VMEM((CH, Co), jnp.float32),
            pltpu.VMEM((CH, Co), jnp.float32),
            pltpu.VMEM((CH, Co), jnp.float32),
            pltpu.VMEM((CSL, 16), jnp.float32),
            pltpu.VMEM((CSL, 16), jnp.float32),
            pltpu.SemaphoreType.DMA,
        ],
    )
    def gather_reduce(gall, idx2d, htf, pmax, pmin, spart, qpart,
                      idxv, rowsv, htv, obmax, obmin, sacc, qacc, sem):
        wid = lax.axis_index("s") * NC + lax.axis_index("c")
        for c in range(CSL):
            sacc[c] = jnp.zeros((16,), jnp.float32)
            qacc[c] = jnp.zeros((16,), jnp.float32)
        pltpu.sync_copy(idx2d.at[pl.ds(wid * WIDXR, WIDXR)], idxv)

        def chunk_body(ch, carry):
            p0 = wid * PTS_W + ch * CH
            pltpu.sync_copy(htf.at[pl.ds(p0, CH)], htv)
            handles = [
                pltpu.async_copy(gall.at[idxv.at[ch * IDXR + r]],
                                 rowsv.at[pl.ds(r * 128, 128)], sem)
                for r in range(IDXR)
            ]
            for h in handles:
                h.wait()

            def point_body(p, carry2):
                base = p * _K
                for c in range(CSL):
                    sl = pl.ds(c * 16, 16)
                    v0 = rowsv[base, sl]

                    def jbody(j, acc):
                        mx, mn, sm, sq = acc
                        v = rowsv[base + j, sl]
                        return (jnp.maximum(mx, v), jnp.minimum(mn, v),
                                sm + v, sq + v * v)

                    mx, mn, sm, sq = lax.fori_loop(
                        1, _K, jbody, (v0, v0, v0, v0 * v0))
                    hv = htv[p, sl]
                    obmax[p, sl] = mx + hv
                    obmin[p, sl] = mn + hv
                    sacc[c] = sacc[c] + sm + float(_K) * hv
                    qacc[c] = qacc[c] + sq + 2.0 * hv * sm \
                        + float(_K) * hv * hv
                return carry2

            lax.fori_loop(0, CH, point_body, 0)
            pltpu.sync_copy(obmax, pmax.at[pl.ds(p0, CH)])
            pltpu.sync_copy(obmin, pmin.at[pl.ds(p0, CH)])
            return carry

        lax.fori_loop(0, NCH, chunk_body, 0)
        pltpu.sync_copy(sacc, spart.at[wid])
        pltpu.sync_copy(qacc, qpart.at[wid])

    return gather_reduce


@jax.jit
def kernel(x, W, gamma, beta):
    B, C, N = x.shape
    Co = W.shape[0]
    R = 256
    T = N // R
    P = B * N

    xt = jnp.transpose(x, (0, 2, 1))
    w1t = jnp.transpose(W[:, :C])
    wd = jnp.transpose(W[:, C:] - W[:, :C])

    idxout, gtout, htout = pl.pallas_call(
        _topk_kernel,
        grid=(B, T),
        in_specs=[
            pl.BlockSpec((1, R, C), lambda b, t: (b, t, 0)),
            pl.BlockSpec((1, C, N), lambda b, t: (b, 0, 0)),
            pl.BlockSpec((C, Co), lambda b, t: (0, 0)),
            pl.BlockSpec((C, Co), lambda b, t: (0, 0)),
        ],
        out_specs=[
            pl.BlockSpec((1, R, _K), lambda b, t: (b, t, 0)),
            pl.BlockSpec((1, R, Co), lambda b, t: (b, t, 0)),
            pl.BlockSpec((1, R, Co), lambda b, t: (b, t, 0)),
        ],
        out_shape=[
            jax.ShapeDtypeStruct((B, N, _K), jnp.int32),
            jax.ShapeDtypeStruct((B, N, Co), jnp.float32),
            jax.ShapeDtypeStruct((B, N, Co), jnp.float32),
        ],
    )(xt, x, w1t, wd)

    info = plsc.get_sparse_core_info()
    NC, NS = info.num_cores, info.num_subcores

    gall = gtout.reshape(P, Co)
    htf = htout.reshape(P, Co)
    idx2d = idxout.reshape(P * _K // 128, 128)

    pmaxf, pminf, spart, qpart = _make_gather_reduce(P, Co, NC, NS)(
        gall, idx2d, htf)

    cnt = jnp.float32(P * _K)
    S = jnp.sum(spart, axis=0).reshape(Co)
    Q = jnp.sum(qpart, axis=0).reshape(Co)
    mean = S / cnt
    var = Q / cnt - mean * mean
    scale = gamma / jnp.sqrt(var + _EPS)
    shift = beta - scale * mean

    out_nc = pl.pallas_call(
        _bn_kernel,
        grid=(B,),
        in_specs=[
            pl.BlockSpec((1, N, Co), lambda b: (b, 0, 0)),
            pl.BlockSpec((1, N, Co), lambda b: (b, 0, 0)),
            pl.BlockSpec((1, Co), lambda b: (0, 0)),
            pl.BlockSpec((1, Co), lambda b: (0, 0)),
        ],
        out_specs=pl.BlockSpec((1, N, Co), lambda b: (b, 0, 0)),
        out_shape=jax.ShapeDtypeStruct((B, N, Co), jnp.float32),
    )(pmaxf.reshape(B, N, Co), pminf.reshape(B, N, Co),
      scale[None], shift[None])

    return jnp.transpose(out_nc, (0, 2, 1))


# speedup vs baseline: 1.1305x; 1.1305x over previous
import functools

import jax
import jax.numpy as jnp
from jax import lax
from jax.experimental import pallas as pl
from jax.experimental.pallas import tpu as pltpu
from jax.experimental.pallas import tpu_sc as plsc

_K = 20
_EPS = 1e-5
_SLOPE = 0.2
_NEG = -3.0e38


def _topk_kernel(xt_rows_ref, x_full_ref, w1t_ref, wd_ref,
                 idx_ref, gt_ref, ht_ref):
    b = pl.program_id(0)

    x_b = x_full_ref[0]
    rows = xt_rows_ref[0]
    R = rows.shape[0]
    N = x_b.shape[1]

    d2 = lax.dot_general(rows, x_b, (((1,), (0,)), ((), ())),
                         preferred_element_type=jnp.float32)
    xxr = jnp.sum(rows * rows, axis=1, keepdims=True)
    xxc = jnp.sum(x_b * x_b, axis=0, keepdims=True)
    negd = 2.0 * d2 - xxr - xxc

    Nh = N // 2
    lft = negd[:, :Nh]
    rgt = negd[:, Nh:]
    swap = rgt > lft
    H = jnp.where(swap, rgt, lft)
    H2 = jnp.where(swap, lft, rgt)
    laneh = lax.broadcasted_iota(jnp.int32, (R, Nh), 1)
    GL = jnp.where(swap, laneh + Nh, laneh)
    GL2 = jnp.where(swap, laneh, laneh + Nh)

    ams = []
    for _ in range(_K):
        m = jnp.max(H, axis=1, keepdims=True)
        cand = jnp.where(H == m, GL, N)
        am = jnp.min(cand, axis=1, keepdims=True)
        ams.append(am)
        eqp = laneh == (am & (Nh - 1))
        H = jnp.where(eqp, H2, H)
        GL = jnp.where(eqp, GL2, GL)
        H2 = jnp.where(eqp, _NEG, H2)

    idx_tile = jnp.concatenate(ams, axis=1) + b * N
    idx_ref[...] = idx_tile[None]

    gt_ref[...] = lax.dot_general(rows, w1t_ref[...], (((1,), (0,)), ((), ())),
                                  preferred_element_type=jnp.float32)[None]
    ht_ref[...] = lax.dot_general(rows, wd_ref[...], (((1,), (0,)), ((), ())),
                                  preferred_element_type=jnp.float32)[None]


def _bn_kernel(pmax_ref, pmin_ref, scale_ref, shift_ref, out_ref):
    scale = scale_ref[...]
    shift = shift_ref[...]
    pick = jnp.where(scale >= 0.0, pmax_ref[0], pmin_ref[0])
    z = pick * scale + shift
    out_ref[...] = jnp.where(z >= 0.0, z, _SLOPE * z)[None]


def _make_gather_reduce(P, Co, NC, NS):
    NW = NC * NS
    PTS_W = P // NW
    CH = 64
    NCH = PTS_W // CH
    ROWS = CH * _K
    IDXR = ROWS // 128
    WIDXR = PTS_W * _K // 128
    CSL = Co // 16

    mesh = plsc.VectorSubcoreMesh(core_axis_name="c", subcore_axis_name="s")

    @functools.partial(
        pl.kernel, mesh=mesh,
        compiler_params=pltpu.CompilerParams(use_tc_tiling_on_sc=False),
        out_type=[
            jax.ShapeDtypeStruct((P, Co), jnp.float32),
            jax.ShapeDtypeStruct((P, Co), jnp.float32),
            jax.ShapeDtypeStruct((NW, CSL, 16), jnp.float32),
            jax.ShapeDtypeStruct((NW, CSL, 16), jnp.float32),
        ],
        scratch_types=[
            pltpu.VMEM((WIDXR, 128), jnp.int32),
            pltpu.VMEM((ROWS, Co), jnp.float32),
            pltpu.VMEM((CH, Co), jnp.float32),
            pltpu.VMEM((CH, Co), jnp.float32),
            pltpu.VMEM((CH, Co), jnp.float32),
            pltpu.VMEM((CSL, 16), jnp.float32),
            pltpu.VMEM((CSL, 16), jnp.float32),
            pltpu.SemaphoreType.DMA,
        ],
    )
    def gather_reduce(gall, idx2d, htf, pmax, pmin, spart, qpart,
                      idxv, rowsv, htv, obmax, obmin, sacc, qacc, sem):
        wid = lax.axis_index("s") * NC + lax.axis_index("c")
        for c in range(CSL):
            sacc[c] = jnp.zeros((16,), jnp.float32)
            qacc[c] = jnp.zeros((16,), jnp.float32)
        pltpu.sync_copy(idx2d.at[pl.ds(wid * WIDXR, WIDXR)], idxv)

        def chunk_body(ch, carry):
            p0 = wid * PTS_W + ch * CH
            pltpu.sync_copy(htf.at[pl.ds(p0, CH)], htv)
            handles = [
                pltpu.async_copy(gall.at[idxv.at[ch * IDXR + r]],
                                 rowsv.at[pl.ds(r * 128, 128)], sem)
                for r in range(IDXR)
            ]
            for h in handles:
                h.wait()

            def point_body(p, carry2):
                base = p * _K
                for c in range(CSL):
                    sl = pl.ds(c * 16, 16)
                    v0 = rowsv[base, sl]

                    def jbody(j, acc):
                        mx, mn, sm, sq = acc
                        v = rowsv[base + j, sl]
                        return (jnp.maximum(mx, v), jnp.minimum(mn, v),
                                sm + v, sq + v * v)

                    mx, mn, sm, sq = lax.fori_loop(
                        1, _K, jbody, (v0, v0, v0, v0 * v0))
                    hv = htv[p, sl]
                    obmax[p, sl] = mx + hv
                    obmin[p, sl] = mn + hv
                    sacc[c] = sacc[c] + sm + float(_K) * hv
                    qacc[c] = qacc[c] + sq + 2.0 * hv * sm \
                        + float(_K) * hv * hv
                return carry2

            lax.fori_loop(0, CH, point_body, 0)
            pltpu.sync_copy(obmax, pmax.at[pl.ds(p0, CH)])
            pltpu.sync_copy(obmin, pmin.at[pl.ds(p0, CH)])
            return carry

        lax.fori_loop(0, NCH, chunk_body, 0)
        pltpu.sync_copy(sacc, spart.at[wid])
        pltpu.sync_copy(qacc, qpart.at[wid])

    return gather_reduce


@jax.jit
def kernel(x, W, gamma, beta):
    B, C, N = x.shape
    Co = W.shape[0]
    R = 256
    T = N // R
    P = B * N

    xt = jnp.transpose(x, (0, 2, 1))
    w1t = jnp.transpose(W[:, :C])
    wd = jnp.transpose(W[:, C:] - W[:, :C])

    idxout, gtout, htout = pl.pallas_call(
        _topk_kernel,
        grid=(B, T),
        in_specs=[
            pl.BlockSpec((1, R, C), lambda b, t: (b, t, 0)),
            pl.BlockSpec((1, C, N), lambda b, t: (b, 0, 0)),
            pl.BlockSpec((C, Co), lambda b, t: (0, 0)),
            pl.BlockSpec((C, Co), lambda b, t: (0, 0)),
        ],
        out_specs=[
            pl.BlockSpec((1, R, _K), lambda b, t: (b, t, 0)),
            pl.BlockSpec((1, R, Co), lambda b, t: (b, t, 0)),
            pl.BlockSpec((1, R, Co), lambda b, t: (b, t, 0)),
        ],
        out_shape=[
            jax.ShapeDtypeStruct((B, N, _K), jnp.int32),
            jax.ShapeDtypeStruct((B, N, Co), jnp.float32),
            jax.ShapeDtypeStruct((B, N, Co), jnp.float32),
        ],
    )(xt, x, w1t, wd)

    info = plsc.get_sparse_core_info()
    NC, NS = info.num_cores, info.num_subcores

    gall = gtout.reshape(P, Co)
    htf = htout.reshape(P, Co)
    idx2d = idxout.reshape(P * _K // 128, 128)

    pmaxf, pminf, spart, qpart = _make_gather_reduce(P, Co, NC, NS)(
        gall, idx2d, htf)

    cnt = jnp.float32(P * _K)
    S = jnp.sum(spart, axis=0).reshape(Co)
    Q = jnp.sum(qpart, axis=0).reshape(Co)
    mean = S / cnt
    var = Q / cnt - mean * mean
    scale = gamma / jnp.sqrt(var + _EPS)
    shift = beta - scale * mean

    out_nc = pl.pallas_call(
        _bn_kernel,
        grid=(B,),
        in_specs=[
            pl.BlockSpec((1, N, Co), lambda b: (b, 0, 0)),
            pl.BlockSpec((1, N, Co), lambda b: (b, 0, 0)),
            pl.BlockSpec((1, Co), lambda b: (0, 0)),
            pl.BlockSpec((1, Co), lambda b: (0, 0)),
        ],
        out_specs=pl.BlockSpec((1, N, Co), lambda b: (b, 0, 0)),
        out_shape=jax.ShapeDtypeStruct((B, N, Co), jnp.float32),
    )(pmaxf.reshape(B, N, Co), pminf.reshape(B, N, Co),
      scale[None], shift[None])

    return jnp.transpose(out_nc, (0, 2, 1))
